# baseline (speedup 1.0000x reference)
import functools

import jax
import jax.numpy as jnp
from jax import lax
from jax.experimental import pallas as pl
from jax.experimental.pallas import tpu as pltpu
from jax.experimental.pallas import tpu_sc as plsc

_NUM_UNITS = 64
_MAX_REL = 128
_L = 2048
_X = 3080


def _rel_pos_sc(table, rv):
    rows = table.shape[0]
    rv_arr = jnp.full((16,), rv, dtype=jnp.int32)
    nc, ns = 2, 16
    nw = nc * ns
    rows_per_worker = _L // nw
    n_chunks = _X // 256
    rem = _X - n_chunks * 256

    mesh = plsc.VectorSubcoreMesh(
        core_axis_name="c", subcore_axis_name="s", num_cores=nc, num_subcores=ns
    )

    @functools.partial(
        pl.kernel,
        out_type=jax.ShapeDtypeStruct((_L, 8, _L // 128, 8, 128), jnp.float32),
        mesh=mesh,
        scratch_types=[
            pltpu.VMEM((16,), jnp.int32),
            pltpu.VMEM((32 * 256,), jnp.int32),
            pltpu.VMEM((32, 256), jnp.float32),
            pltpu.VMEM_SHARED((8, _NUM_UNITS, _X), jnp.float32),
            pltpu.SemaphoreType.DMA,
            pltpu.SemaphoreType.DMA,
        ],
        compiler_params=pltpu.CompilerParams(use_tc_tiling_on_sc=False),
    )
    def k(table_hbm, rv_hbm, out_hbm, rv_v, idx2_v, tg_v, et8_sh, sem, sem0):
        c = lax.axis_index("c")
        s = lax.axis_index("s")
        lane16 = lax.iota(jnp.int32, 16)
        org = (1 - c) * 1024

        pltpu.sync_copy(rv_hbm, rv_v)
        rv_vec = rv_v[...]
        p = s % 8
        h = s // 8
        u0 = h * 32

        def cc_body(cc, carry):
            def ul_body(ul, carry2):
                u = u0 + ul

                def rc_body(rc, c2):
                    x = cc * 256 + rc * 16 + lane16
                    kk = org + p + x
                    idx = (
                        jnp.clip(kk - (_L - 1), -_MAX_REL, _MAX_REL)
                        + _MAX_REL + rv_vec
                    )
                    idx = jnp.clip(idx, 0, rows - 1)
                    idx2_v[pl.ds(ul * 256 + rc * 16, 16)] = (
                        idx * _NUM_UNITS + u
                    )
                    return c2

                return lax.fori_loop(0, 16, rc_body, carry2)

            lax.fori_loop(0, 32, ul_body, 0)

            def gb_body(gb, c3):
                descs = []
                for gg in range(16):
                    g = gb * 16 + gg
                    descs.append(
                        pltpu.async_copy(
                            table_hbm.at[idx2_v.at[pl.ds(g * 128, 128)]],
                            tg_v.at[g // 2, pl.ds((g % 2) * 128, 128)],
                            sem,
                        )
                    )
                for d in descs:
                    d.wait()
                return c3

            lax.fori_loop(0, 4, gb_body, 0)
            pltpu.sync_copy(
                tg_v, et8_sh.at[p, pl.ds(u0, 32), pl.ds(cc * 256, 256)]
            )
            return carry

        lax.fori_loop(0, n_chunks, cc_body, 0)

        def ul_rem(ul, carry):
            u = u0 + ul
            x = n_chunks * 256 + lane16 % rem
            kk = org + p + x
            idx = (
                jnp.clip(kk - (_L - 1), -_MAX_REL, _MAX_REL)
                + _MAX_REL + rv_vec
            )
            idx = jnp.clip(idx, 0, rows - 1)
            idx2_v[pl.ds(ul * 16, 16)] = idx * _NUM_UNITS + u
            return carry

        lax.fori_loop(0, 32, ul_rem, 0)
        for g in range(4):
            pltpu.async_copy(
                table_hbm.at[idx2_v.at[pl.ds(g * 128, 128)]],
                tg_v.at[g, pl.ds(0, 128)],
                sem,
            ).wait()

        def ul_rem2(ul, carry):
            pltpu.sync_copy(
                tg_v.at[ul // 8, pl.ds((ul % 8) * 16, 8)],
                et8_sh.at[p, u0 + ul, pl.ds(n_chunks * 256, rem)],
            )
            return carry

        lax.fori_loop(0, 32, ul_rem2, 0)
        plsc.subcore_barrier()

        w = c * ns + s
        ibase = w * rows_per_worker
        tiles_per_row = 8 * (_L // 128)

        def i_body(iw, carry):
            i = ibase + iw
            wcol = (_L - 1) - i
            ph = wcol % 8
            x0 = pl.multiple_of(wcol - ph - org, 8)

            def tu_body(tu, c1):
                def tj_body(tj, c2):
                    pltpu.async_copy(
                        et8_sh.at[
                            ph,
                            pl.ds(pl.multiple_of(tu * 8, 8), 8),
                            pl.ds(pl.multiple_of(x0 + tj * 128, 8), 128),
                        ],
                        out_hbm.at[i, tu, tj],
                        sem0,
                    )
                    return c2

                return lax.fori_loop(0, _L // 128, tj_body, c1)

            lax.fori_loop(0, 8, tu_body, 0)

            @pl.when(iw > 0)
            def _drain_prev():
                def d_body(d, c3):
                    pltpu.make_async_copy(
                        et8_sh.at[0, pl.ds(0, 8), pl.ds(0, 128)],
                        out_hbm.at[0, 0, 0],
                        sem0,
                    ).wait()
                    return c3

                lax.fori_loop(0, tiles_per_row, d_body, 0)

            return carry

        lax.fori_loop(0, rows_per_worker, i_body, 0)

        def d_last(d, c3):
            pltpu.make_async_copy(
                et8_sh.at[0, pl.ds(0, 8), pl.ds(0, 128)],
                out_hbm.at[0, 0, 0],
                sem0,
            ).wait()
            return c3

        lax.fori_loop(0, tiles_per_row, d_last, 0)

    return k(table.reshape(-1), rv_arr)


def kernel(embeddings_table, length_q, length_k, relative_v):
    out5 = _rel_pos_sc(embeddings_table, relative_v)
    return out5.transpose(0, 2, 4, 1, 3).reshape(_L, _L, _NUM_UNITS)


# speedup vs baseline: 7.6987x; 7.6987x over previous
import functools

import jax
import jax.numpy as jnp
from jax import lax
from jax.experimental import pallas as pl
from jax.experimental.pallas import tpu as pltpu
from jax.experimental.pallas import tpu_sc as plsc

_NUM_UNITS = 64
_MAX_REL = 128
_L = 2048
_NI = 16
_W = _L + _NI


def _rel_pos_sc(table, rv):
    rows = table.shape[0]
    rv_arr = jnp.full((16,), rv, dtype=jnp.int32)
    e_rows = 2 * _L
    nc, ns = 2, 16
    nw = nc * ns
    k_per_tile = e_rows // ns
    n_ug = _NUM_UNITS // 8
    ntasks = (_L // _NI) * n_ug
    tasks_per_worker = ntasks // nw

    mesh = plsc.VectorSubcoreMesh(
        core_axis_name="c", subcore_axis_name="s", num_cores=nc, num_subcores=ns
    )

    @functools.partial(
        pl.kernel,
        out_type=jax.ShapeDtypeStruct((_L, 8, _L // 128, 8, 128), jnp.float32),
        mesh=mesh,
        scratch_types=[
            pltpu.VMEM((16,), jnp.int32),
            pltpu.VMEM((k_per_tile,), jnp.int32),
            pltpu.VMEM((_NUM_UNITS * k_per_tile,), jnp.int32),
            pltpu.VMEM((_NUM_UNITS, k_per_tile), jnp.float32),
            pltpu.VMEM((8, _W), jnp.float32),
            pltpu.VMEM((_L // 128, 8, 128), jnp.float32),
            pltpu.VMEM((_L // 128, 8, 128), jnp.float32),
            pltpu.VMEM_SHARED((_NUM_UNITS, e_rows), jnp.float32),
            pltpu.SemaphoreType.DMA,
            pltpu.SemaphoreType.DMA,
            pltpu.SemaphoreType.DMA,
        ],
        compiler_params=pltpu.CompilerParams(use_tc_tiling_on_sc=False),
    )
    def k(table_hbm, rv_hbm, out_hbm, rv_v, idx_v, idx2_v, tg_v, win_v, buf0_v,
          buf1_v, et_sh, sem, sem0, sem1):
        c = lax.axis_index("c")
        s = lax.axis_index("s")
        lane16 = lax.iota(jnp.int32, 16)

        pltpu.sync_copy(rv_hbm, rv_v)
        rv_vec = rv_v[...]
        k0 = s * k_per_tile
        for ch in range(k_per_tile // 16):
            kk = lane16 + (k0 + ch * 16)
            idx = jnp.clip(kk - (_L - 1), -_MAX_REL, _MAX_REL) + _MAX_REL + rv_vec
            idx_v[pl.ds(ch * 16, 16)] = jnp.clip(idx, 0, rows - 1)

        def u_body(u, carry):
            def rc_body(rc, c2):
                flat = idx_v[pl.ds(rc * 16, 16)] * _NUM_UNITS + u
                idx2_v[pl.ds(u * k_per_tile + rc * 16, 16)] = flat
                return c2

            return lax.fori_loop(0, k_per_tile // 16, rc_body, carry)

        lax.fori_loop(0, _NUM_UNITS, u_body, 0)

        def gb_body(gb, carry):
            descs = []
            for gg in range(16):
                g = gb * 16 + gg
                descs.append(
                    pltpu.async_copy(
                        table_hbm.at[idx2_v.at[pl.ds(g * 128, 128)]],
                        tg_v.at[gb * 8 + gg // 2, pl.ds((gg % 2) * 128, 128)],
                        sem,
                    )
                )
            for d in descs:
                d.wait()
            return carry

        lax.fori_loop(0, _NUM_UNITS * k_per_tile // (16 * 128), gb_body, 0)
        pltpu.sync_copy(tg_v, et_sh.at[:, pl.ds(k0, k_per_tile)])
        plsc.subcore_barrier()

        w = c * ns + s

        bufs = (buf0_v, buf1_v)
        sems = (sem0, sem1)

        def task_body(tl, carry):
            tid = w * tasks_per_worker + tl
            i0 = (tid // n_ug) * _NI
            tu = tid % n_ug
            u0 = tu * 8
            wlo = (_L - 1) - (i0 + _NI - 1)
            pltpu.sync_copy(et_sh.at[pl.ds(u0, 8), pl.ds(wlo, _W)], win_v)

            def ii2_body(ii2, c1):
                for b in range(2):
                    ii = ii2 * 2 + b
                    i = i0 + ii
                    off = (_NI - 1) - ii
                    buf = bufs[b]
                    bsem = sems[b]

                    @pl.when(ii2 > 0)
                    def _wait():
                        pltpu.make_async_copy(buf, out_hbm.at[i, tu], bsem).wait()

                    for su in range(8):

                        @functools.partial(plsc.parallel_loop, 0, _L // 128,
                                           unroll=4)
                        def _asm(tj):
                            base = off + tj * 128
                            for lc in range(8):
                                buf[tj, su, pl.ds(lc * 16, 16)] = (
                                    win_v[su, pl.ds(base + lc * 16, 16)]
                                )

                    pltpu.async_copy(buf, out_hbm.at[i, tu], bsem)
                return c1

            lax.fori_loop(0, _NI // 2, ii2_body, 0)
            pltpu.make_async_copy(buf0_v, out_hbm.at[i0 + _NI - 2, tu], sem0).wait()
            pltpu.make_async_copy(buf1_v, out_hbm.at[i0 + _NI - 1, tu], sem1).wait()
            return carry

        lax.fori_loop(0, tasks_per_worker, task_body, 0)

    return k(table.reshape(-1), rv_arr)


def kernel(embeddings_table, length_q, length_k, relative_v):
    out5 = _rel_pos_sc(embeddings_table, relative_v)
    return out5.transpose(0, 2, 4, 1, 3).reshape(_L, _L, _NUM_UNITS)
